# Initial kernel scaffold; baseline (speedup 1.0000x reference)
import jax
import jax.numpy as jnp
from jax.experimental import pallas as pl


def kernel(q, k, embeddings):
    raise NotImplementedError("write your pallas kernel here")


# speedup vs baseline: 8.1342x; 8.1342x over previous
import functools

import jax
import jax.numpy as jnp
from jax import lax
from jax.experimental import pallas as pl
from jax.experimental.pallas import tpu as pltpu
from jax.experimental.pallas import tpu_sc as plsc


def _build_sc_kernel(q_len, k_len, input_dim, output_dim, dtype):
    info = plsc.get_sparse_core_info()
    nc, ns, nl = info.num_cores, info.num_subcores, info.num_lanes
    nw = nc * ns
    assert q_len % nw == 0
    rows_per_w = q_len // nw
    max_index = (input_dim - 1) // 2
    emb_words = input_dim * output_dim
    row_words = k_len * output_dim
    band_rows = rows_per_w - 1 + k_len + 1
    band_words = band_rows * output_dim
    mesh = plsc.VectorSubcoreMesh(core_axis_name="c", subcore_axis_name="s")

    @functools.partial(
        pl.kernel,
        mesh=mesh,
        out_type=jax.ShapeDtypeStruct((q_len * row_words,), dtype),
        scratch_types=[
            pltpu.VMEM((emb_words,), dtype),
            pltpu.VMEM((band_words,), dtype),
            pltpu.SemaphoreType.DMA,
        ],
    )
    def sc_kernel(emb_hbm, out_hbm, emb_ref, band_ref, osem):
        wid = lax.axis_index("s") * nc + lax.axis_index("c")
        i0 = wid * rows_per_w
        s0 = i0 + (rows_per_w - 1) + max_index

        pltpu.sync_copy(emb_hbm, emb_ref)

        def build(u, _):
            e = jnp.clip(s0 - u, 0, 2 * max_index) * output_dim
            b = u * output_dim
            band_ref[pl.ds(b, nl)] = emb_ref[pl.ds(e, nl)]
            band_ref[pl.ds(b + nl, nl)] = emb_ref[pl.ds(e + nl, nl)]
            return _

        lax.fori_loop(0, band_rows, build, 0)

        depth = 4

        def start(r):
            return pltpu.async_copy(
                band_ref.at[pl.ds((rows_per_w - 1 - r) * output_dim, row_words)],
                out_hbm.at[pl.ds((i0 + r) * row_words, row_words)],
                osem,
            )

        primed = [start(r) for r in range(depth)]

        def body(r, _):
            cp = start(r)
            cp.wait()
            return _

        lax.fori_loop(depth, rows_per_w, body, 0)
        for cp in primed:
            cp.wait()

    return sc_kernel


def kernel(q, k, embeddings):
    q_len = q.shape[1]
    k_len = k.shape[1]
    input_dim, output_dim = embeddings.shape
    fn = _build_sc_kernel(q_len, k_len, input_dim, output_dim, embeddings.dtype)
    out = fn(embeddings.reshape(-1))
    return out.reshape(q_len, k_len, output_dim)


# speedup vs baseline: 8.7124x; 1.0711x over previous
import functools

import jax
import jax.numpy as jnp
from jax import lax
from jax.experimental import pallas as pl
from jax.experimental.pallas import tpu as pltpu
from jax.experimental.pallas import tpu_sc as plsc


def _build_sc_kernel(q_len, k_len, input_dim, output_dim, dtype):
    info = plsc.get_sparse_core_info()
    nc, ns, nl = info.num_cores, info.num_subcores, info.num_lanes
    nw = nc * ns
    assert q_len % nw == 0
    rows_per_w = q_len // nw
    max_index = (input_dim - 1) // 2
    emb_words = input_dim * output_dim
    row_words = k_len * output_dim
    band_rows = rows_per_w - 1 + k_len + 1
    band_words = band_rows * output_dim
    mesh = plsc.VectorSubcoreMesh(core_axis_name="c", subcore_axis_name="s")

    j_chunk = 512
    n_j = k_len // j_chunk
    win_rows = rows_per_w + j_chunk

    @functools.partial(
        pl.kernel,
        mesh=mesh,
        out_type=jax.ShapeDtypeStruct((q_len, k_len, output_dim), dtype),
        scratch_types=[
            pltpu.VMEM((emb_words,), dtype),
            pltpu.VMEM((win_rows, output_dim), dtype),
            pltpu.SemaphoreType.DMA,
        ],
    )
    def sc_kernel(emb_hbm, out_hbm, emb_ref, band_ref, osem):
        wid = lax.axis_index("s") * nc + lax.axis_index("c")
        i0 = wid * rows_per_w
        s0 = i0 + (rows_per_w - 1) + max_index

        pltpu.sync_copy(emb_hbm, emb_ref)

        depth = 4

        def jloop(jc, _):
            base = s0 - jc * j_chunk

            def build(u, _):
                e = jnp.clip(base - u, 0, 2 * max_index) * output_dim
                band_ref[u, pl.ds(0, nl)] = emb_ref[pl.ds(e, nl)]
                band_ref[u, pl.ds(nl, nl)] = emb_ref[pl.ds(e + nl, nl)]
                return _

            lax.fori_loop(0, win_rows, build, 0)

            def start(r):
                return pltpu.async_copy(
                    band_ref.at[pl.ds(rows_per_w - 1 - r, j_chunk)],
                    out_hbm.at[i0 + r, pl.ds(jc * j_chunk, j_chunk)],
                    osem,
                )

            primed = [start(r) for r in range(depth)]

            def body(r, _):
                cp = start(r)
                cp.wait()
                return _

            lax.fori_loop(depth, rows_per_w, body, 0)
            for cp in primed:
                cp.wait()
            return _

        lax.fori_loop(0, n_j, jloop, 0)

    return sc_kernel


def kernel(q, k, embeddings):
    q_len = q.shape[1]
    k_len = k.shape[1]
    input_dim, output_dim = embeddings.shape
    fn = _build_sc_kernel(q_len, k_len, input_dim, output_dim, embeddings.dtype)
    return fn(embeddings.reshape(-1))


# speedup vs baseline: 54.8933x; 6.3006x over previous
import functools

import jax
import jax.numpy as jnp
from jax import lax
from jax.experimental import pallas as pl
from jax.experimental.pallas import tpu as pltpu
from jax.experimental.pallas import tpu_sc as plsc


def _build_sc_band(q_len, k_len, input_dim, output_dim, dtype, band_cols):
    info = plsc.get_sparse_core_info()
    nc, ns, nl = info.num_cores, info.num_subcores, info.num_lanes
    nw = nc * ns
    assert band_cols % nw == 0 and output_dim == 2 * nl
    t_per_w = band_cols // nw
    max_index = (input_dim - 1) // 2
    s0 = (q_len - 1) + max_index
    emb_words = input_dim * output_dim
    seg_words = t_per_w * output_dim
    mesh = plsc.VectorSubcoreMesh(core_axis_name="c", subcore_axis_name="s")

    @functools.partial(
        pl.kernel,
        mesh=mesh,
        out_type=jax.ShapeDtypeStruct((band_cols * output_dim,), dtype),
        scratch_types=[
            pltpu.VMEM((emb_words,), dtype),
            pltpu.VMEM((seg_words,), dtype),
        ],
    )
    def sc_kernel(emb_hbm, band_hbm, emb_ref, seg_ref):
        wid = lax.axis_index("s") * nc + lax.axis_index("c")
        t0 = wid * t_per_w
        pltpu.sync_copy(emb_hbm, emb_ref)

        def build(t, _):
            e = jnp.clip(s0 - (t0 + t), 0, 2 * max_index) * output_dim
            b = t * output_dim
            seg_ref[pl.ds(b, nl)] = emb_ref[pl.ds(e, nl)]
            seg_ref[pl.ds(b + nl, nl)] = emb_ref[pl.ds(e + nl, nl)]
            return _

        lax.fori_loop(0, t_per_w, build, 0)
        pltpu.sync_copy(seg_ref, band_hbm.at[pl.ds(wid * seg_words, seg_words)])

    return sc_kernel


def _build_tc_slide(q_len, k_len, output_dim, dtype, band_cols, planes_per_blk):
    grid = q_len // planes_per_blk

    def body(band_ref, out_ref):
        p = pl.program_id(0)
        band = band_ref[...]
        i_last = p * planes_per_blk + (planes_per_blk - 1)
        t_last = (q_len - 1) - i_last
        shift = lax.rem(band_cols - t_last, band_cols)
        rolled = pltpu.roll(band, shift, axis=1)
        for r in range(planes_per_blk):
            off = planes_per_blk - 1 - r
            out_ref[r] = rolled[:, off:off + k_len]

    return pl.pallas_call(
        body,
        grid=(grid,),
        in_specs=[pl.BlockSpec((output_dim, band_cols), lambda p: (0, 0))],
        out_specs=pl.BlockSpec(
            (planes_per_blk, output_dim, k_len), lambda p: (p, 0, 0)
        ),
        out_shape=jax.ShapeDtypeStruct((q_len, output_dim, k_len), dtype),
    )


def kernel(q, k, embeddings):
    q_len = q.shape[1]
    k_len = k.shape[1]
    input_dim, output_dim = embeddings.shape
    band_cols = ((q_len + k_len - 1 + 127) // 128) * 128
    dtype = embeddings.dtype
    band_flat = _build_sc_band(
        q_len, k_len, input_dim, output_dim, dtype, band_cols
    )(embeddings.reshape(-1))
    band = band_flat.reshape(band_cols, output_dim).T
    out_t = _build_tc_slide(q_len, k_len, output_dim, dtype, band_cols, 8)(band)
    return out_t.transpose(0, 2, 1)


# speedup vs baseline: 68.2183x; 1.2427x over previous
import functools

import jax
import jax.numpy as jnp
from jax import lax
from jax.experimental import pallas as pl
from jax.experimental.pallas import tpu as pltpu
from jax.experimental.pallas import tpu_sc as plsc


def _build_sc_band(q_len, k_len, input_dim, output_dim, dtype, band_cols):
    info = plsc.get_sparse_core_info()
    nc, ns, nl = info.num_cores, info.num_subcores, info.num_lanes
    nw = nc * ns
    assert band_cols % nw == 0 and output_dim == 2 * nl
    t_per_w = band_cols // nw
    max_index = (input_dim - 1) // 2
    s0 = (q_len - 1) + max_index
    emb_words = input_dim * output_dim
    seg_words = t_per_w * output_dim
    mesh = plsc.VectorSubcoreMesh(core_axis_name="c", subcore_axis_name="s")

    @functools.partial(
        pl.kernel,
        mesh=mesh,
        out_type=jax.ShapeDtypeStruct((band_cols * output_dim,), dtype),
        scratch_types=[
            pltpu.VMEM((emb_words,), dtype),
            pltpu.VMEM((seg_words,), dtype),
        ],
    )
    def sc_kernel(emb_hbm, band_hbm, emb_ref, seg_ref):
        wid = lax.axis_index("s") * nc + lax.axis_index("c")
        t0 = wid * t_per_w
        pltpu.sync_copy(emb_hbm, emb_ref)

        def build(t, _):
            e = jnp.clip(s0 - (t0 + t), 0, 2 * max_index) * output_dim
            b = t * output_dim
            seg_ref[pl.ds(b, nl)] = emb_ref[pl.ds(e, nl)]
            seg_ref[pl.ds(b + nl, nl)] = emb_ref[pl.ds(e + nl, nl)]
            return _

        lax.fori_loop(0, t_per_w, build, 0)
        pltpu.sync_copy(seg_ref, band_hbm.at[pl.ds(wid * seg_words, seg_words)])

    return sc_kernel


def _build_tc_slide(q_len, k_len, output_dim, dtype, band_cols, planes_per_blk):
    grid = q_len // planes_per_blk

    def body(band_ref, out_ref):
        p = pl.program_id(0)
        band = band_ref[...]
        i_last = p * planes_per_blk + (planes_per_blk - 1)
        t_last = (q_len - 1) - i_last
        shift = lax.rem(band_cols - t_last, band_cols)
        rolled = pltpu.roll(band, shift, axis=1)
        for r in range(planes_per_blk):
            off = planes_per_blk - 1 - r
            out_ref[r] = rolled[:, off:off + k_len]

    return pl.pallas_call(
        body,
        grid=(grid,),
        in_specs=[pl.BlockSpec((output_dim, band_cols), lambda p: (0, 0))],
        out_specs=pl.BlockSpec(
            (planes_per_blk, output_dim, k_len), lambda p: (p, 0, 0)
        ),
        out_shape=jax.ShapeDtypeStruct((q_len, output_dim, k_len), dtype),
    )


def kernel(q, k, embeddings):
    q_len = q.shape[1]
    k_len = k.shape[1]
    input_dim, output_dim = embeddings.shape
    band_cols = ((q_len + k_len - 1 + 127) // 128) * 128
    dtype = embeddings.dtype
    band_flat = _build_sc_band(
        q_len, k_len, input_dim, output_dim, dtype, band_cols
    )(embeddings.reshape(-1))
    band = band_flat.reshape(band_cols, output_dim).T
    out_t = _build_tc_slide(q_len, k_len, output_dim, dtype, band_cols, 16)(band)
    return out_t.transpose(0, 2, 1)


# speedup vs baseline: 77.3255x; 1.1335x over previous
import functools

import jax
import jax.numpy as jnp
from jax import lax
from jax.experimental import pallas as pl
from jax.experimental.pallas import tpu as pltpu
from jax.experimental.pallas import tpu_sc as plsc


def _build_sc_band(q_len, k_len, input_dim, output_dim, dtype, band_cols):
    info = plsc.get_sparse_core_info()
    nc, ns, nl = info.num_cores, info.num_subcores, info.num_lanes
    nw = nc * ns
    assert band_cols % nw == 0 and output_dim == 2 * nl
    t_per_w = band_cols // nw
    max_index = (input_dim - 1) // 2
    s0 = (q_len - 1) + max_index
    emb_words = input_dim * output_dim
    seg_words = t_per_w * output_dim
    mesh = plsc.VectorSubcoreMesh(core_axis_name="c", subcore_axis_name="s")

    @functools.partial(
        pl.kernel,
        mesh=mesh,
        out_type=jax.ShapeDtypeStruct((band_cols * output_dim,), dtype),
        scratch_types=[
            pltpu.VMEM((emb_words,), dtype),
            pltpu.VMEM((seg_words,), dtype),
        ],
    )
    def sc_kernel(emb_hbm, band_hbm, emb_ref, seg_ref):
        wid = lax.axis_index("s") * nc + lax.axis_index("c")
        t0 = wid * t_per_w
        pltpu.sync_copy(emb_hbm, emb_ref)

        def build(t, _):
            e = jnp.clip(s0 - (t0 + t), 0, 2 * max_index) * output_dim
            b = t * output_dim
            seg_ref[pl.ds(b, nl)] = emb_ref[pl.ds(e, nl)]
            seg_ref[pl.ds(b + nl, nl)] = emb_ref[pl.ds(e + nl, nl)]
            return _

        lax.fori_loop(0, t_per_w, build, 0)
        pltpu.sync_copy(seg_ref, band_hbm.at[pl.ds(wid * seg_words, seg_words)])

    return sc_kernel


def _build_tc_slide(q_len, k_len, output_dim, dtype, band_cols, planes_per_blk):
    grid = q_len // planes_per_blk

    def body(band_ref, out_ref):
        p = pl.program_id(0)
        band = band_ref[...]
        i_last = p * planes_per_blk + (planes_per_blk - 1)
        t_last = (q_len - 1) - i_last
        shift = lax.rem(band_cols - t_last, band_cols)
        rolled = pltpu.roll(band, shift, axis=1)
        for r in range(planes_per_blk):
            off = planes_per_blk - 1 - r
            out_ref[r] = rolled[:, off:off + k_len]

    return pl.pallas_call(
        body,
        grid=(grid,),
        in_specs=[pl.BlockSpec((output_dim, band_cols), lambda p: (0, 0))],
        out_specs=pl.BlockSpec(
            (planes_per_blk, output_dim, k_len), lambda p: (p, 0, 0)
        ),
        out_shape=jax.ShapeDtypeStruct((q_len, output_dim, k_len), dtype),
    )


def kernel(q, k, embeddings):
    q_len = q.shape[1]
    k_len = k.shape[1]
    input_dim, output_dim = embeddings.shape
    band_cols = ((q_len + k_len - 1 + 127) // 128) * 128
    dtype = embeddings.dtype
    band_flat = _build_sc_band(
        q_len, k_len, input_dim, output_dim, dtype, band_cols
    )(embeddings.reshape(-1))
    band = band_flat.reshape(band_cols, output_dim).T
    out_t = _build_tc_slide(q_len, k_len, output_dim, dtype, band_cols, 32)(band)
    return out_t.transpose(0, 2, 1)
